# baseline (speedup 1.0000x reference)
import functools

import jax
import jax.numpy as jnp
from jax import lax
from jax.experimental import pallas as pl
from jax.experimental.pallas import tpu as pltpu
from jax.experimental.pallas import tpu_sc as plsc

BATCH = 16384
FIELDS = 26
EMBED_DIM = 32
N = BATCH * FIELDS
NC, NS = 2, 16
NW = NC * NS
RPW = N // NW
CHUNK = 128
CPW = RPW // CHUNK
L = 16
TBL_WIDE = 250000


def _sc_gather(qidx, r32, table_w):
    mesh = plsc.VectorSubcoreMesh(
        core_axis_name="c", subcore_axis_name="s",
        num_cores=NC, num_subcores=NS)

    @functools.partial(
        pl.kernel,
        out_type=jax.ShapeDtypeStruct((N, EMBED_DIM), jnp.float32),
        mesh=mesh,
        scratch_types=[
            pltpu.VMEM((CPW, CHUNK), jnp.int32),
            pltpu.VMEM((CPW, CHUNK), jnp.int32),
            pltpu.VMEM((CHUNK, 128), jnp.float32),
            pltpu.VMEM((CHUNK, 128), jnp.float32),
            pltpu.VMEM((CHUNK, EMBED_DIM), jnp.float32),
            pltpu.VMEM((CHUNK, EMBED_DIM), jnp.float32),
            pltpu.SemaphoreType.DMA,
            pltpu.SemaphoreType.DMA,
            pltpu.SemaphoreType.DMA,
            pltpu.SemaphoreType.DMA,
        ],
        compiler_params=pltpu.CompilerParams(needs_layout_passes=False),
    )
    def k(q_hbm, r_hbm, tab_hbm, out_hbm,
          q_v, r_v, w0, w1, e0, e1, sg0, sg1, sw0, sw1):
        wid = lax.axis_index("s") * NC + lax.axis_index("c")
        pltpu.sync_copy(q_hbm.at[pl.ds(wid * CPW, CPW)], q_v)
        pltpu.sync_copy(r_hbm.at[pl.ds(wid * CPW, CPW)], r_v)
        w = (w0, w1)
        e = (e0, e1)
        sg = (sg0, sg1)
        sw = (sw0, sw1)

        def fire_g(b, c):
            pltpu.async_copy(tab_hbm.at[q_v.at[c]], w[b], sg[b])

        def drain_g(b):
            pltpu.make_async_copy(tab_hbm.at[q_v.at[0]], w[b], sg[b]).wait()

        def extract(b, c):
            for i0 in range(0, CHUNK, L):
                rowv = i0 + lax.iota(jnp.int32, L)
                rv = r_v[c, pl.ds(i0, L)]

                @pl.loop(0, EMBED_DIM)
                def col(j):
                    v = plsc.load_gather(w[b], [rowv, rv + j])
                    plsc.store_scatter(e[b], [rowv, jnp.zeros((L,), jnp.int32) + j], v)

        def fire_w(b, c):
            pltpu.async_copy(
                e[b], out_hbm.at[pl.ds(wid * RPW + c * CHUNK, CHUNK)], sw[b])

        def wait_w(b):
            pltpu.make_async_copy(
                e[b], out_hbm.at[pl.ds(wid * RPW, CHUNK)], sw[b]).wait()

        fire_g(0, 0)
        fire_g(1, 1)

        @pl.loop(0, CPW // 2 - 1)
        def body(i):
            c = i * 2
            drain_g(0)
            extract(0, c)
            fire_w(0, c)
            fire_g(0, c + 2)
            drain_g(1)
            extract(1, c + 1)
            fire_w(1, c + 1)
            fire_g(1, c + 3)
            wait_w(0)
            wait_w(1)

        drain_g(0)
        extract(0, CPW - 2)
        fire_w(0, CPW - 2)
        drain_g(1)
        extract(1, CPW - 1)
        fire_w(1, CPW - 1)
        wait_w(0)
        wait_w(1)

    return k(qidx, r32, table_w)


def kernel(inputs, table):
    flat = inputs.astype(jnp.int32).reshape(N // CHUNK, CHUNK)
    qidx = flat >> 2
    r32 = (flat & 3) << 5
    table_w = table.reshape(TBL_WIDE, 128)
    out = _sc_gather(qidx, r32, table_w)
    return out.reshape(BATCH, FIELDS, EMBED_DIM)


# speedup vs baseline: 1.6986x; 1.6986x over previous
import functools

import jax
import jax.numpy as jnp
from jax import lax
from jax.experimental import pallas as pl
from jax.experimental.pallas import tpu as pltpu
from jax.experimental.pallas import tpu_sc as plsc

BATCH = 16384
FIELDS = 26
EMBED_DIM = 32
NC, NS = 2, 16
NW = NC * NS
BPW = BATCH // NW
NB = 32
STEPS = BPW // NB


def _sc_gather(idx, table):
    mesh = plsc.VectorSubcoreMesh(
        core_axis_name="c", subcore_axis_name="s",
        num_cores=NC, num_subcores=NS)

    @functools.partial(
        pl.kernel,
        out_type=jax.ShapeDtypeStruct((BATCH, FIELDS, EMBED_DIM), jnp.float32),
        mesh=mesh,
        scratch_types=[
            pltpu.VMEM((BPW, FIELDS), jnp.int32),
            pltpu.VMEM((NB, FIELDS, EMBED_DIM), jnp.float32),
            pltpu.VMEM((NB, FIELDS, EMBED_DIM), jnp.float32),
            pltpu.SemaphoreType.DMA,
            pltpu.SemaphoreType.DMA,
            pltpu.SemaphoreType.DMA,
            pltpu.SemaphoreType.DMA,
        ],
        compiler_params=pltpu.CompilerParams(use_tc_tiling_on_sc=False),
    )
    def k(idx_hbm, table_hbm, out_hbm, idx_v, rows0, rows1, sg0, sg1, sw0, sw1):
        wid = lax.axis_index("s") * NC + lax.axis_index("c")
        b0 = wid * BPW
        pltpu.sync_copy(idx_hbm.at[pl.ds(b0, BPW)], idx_v)
        rows = (rows0, rows1)
        sg = (sg0, sg1)
        sw = (sw0, sw1)

        def fire_g(b, t):
            for s in range(NB):
                pltpu.async_copy(
                    table_hbm.at[idx_v.at[t * NB + s]], rows[b].at[s], sg[b])

        def drain_g(b):
            for s in range(NB):
                pltpu.make_async_copy(
                    table_hbm.at[idx_v.at[s]], rows[b].at[s], sg[b]).wait()

        def fire_w(b, t):
            pltpu.async_copy(
                rows[b], out_hbm.at[pl.ds(b0 + t * NB, NB)], sw[b])

        def wait_w(b):
            pltpu.make_async_copy(
                rows[b], out_hbm.at[pl.ds(b0, NB)], sw[b]).wait()

        fire_g(0, 0)
        fire_g(1, 1)

        @pl.loop(0, STEPS // 2 - 1)
        def body(i):
            t = i * 2
            drain_g(0)
            fire_w(0, t)
            drain_g(1)
            fire_w(1, t + 1)
            wait_w(0)
            fire_g(0, t + 2)
            wait_w(1)
            fire_g(1, t + 3)

        drain_g(0)
        fire_w(0, STEPS - 2)
        drain_g(1)
        fire_w(1, STEPS - 1)
        wait_w(0)
        wait_w(1)

    return k(idx, table)


def kernel(inputs, table):
    return _sc_gather(inputs.astype(jnp.int32), table)
